# baseline (speedup 1.0000x reference)
import functools

import jax
import jax.numpy as jnp
from jax import lax
from jax.experimental import pallas as pl
from jax.experimental.pallas import tpu as pltpu
from jax.experimental.pallas import tpu_sc as plsc

_N = 10000
_E = 320000
_D = 128
_NC, _NS = 2, 16
_NW = _NC * _NS
_EPW = _E // _NW
_L = 16
_UNROLL = 5
_CH0 = 4800


def _sc_ego_weights(ei_flat, wts):
    mesh = plsc.VectorSubcoreMesh(
        core_axis_name="c", subcore_axis_name="s",
        num_cores=_NC, num_subcores=_NS)

    @functools.partial(
        pl.kernel,
        out_type=jax.ShapeDtypeStruct((_NW, _N), jnp.float32),
        mesh=mesh,
        compiler_params=pltpu.CompilerParams(needs_layout_passes=False),
        scratch_types=[
            pltpu.VMEM((_EPW,), jnp.int32),
            pltpu.VMEM((_EPW,), jnp.int32),
            pltpu.VMEM((_EPW,), jnp.float32),
            pltpu.VMEM((_N,), jnp.float32),
            pltpu.SemaphoreType.DMA,
            pltpu.SemaphoreType.DMA,
            pltpu.SemaphoreType.DMA,
        ],
    )
    def k(ei_hbm, w_hbm, out_hbm, src_v, dst_v, w_v, acc_v, s0, s1, s2):
        wid = lax.axis_index("s") * _NC + lax.axis_index("c")
        base = wid * _EPW

        def fetch(lo, n):
            return (
                pltpu.async_copy(ei_hbm.at[pl.ds(base + lo, n)],
                                 src_v.at[pl.ds(lo, n)], s0),
                pltpu.async_copy(ei_hbm.at[pl.ds(_E + base + lo, n)],
                                 dst_v.at[pl.ds(lo, n)], s1),
                pltpu.async_copy(w_hbm.at[pl.ds(base + lo, n)],
                                 w_v.at[pl.ds(lo, n)], s2),
            )

        def scan(lo, n):
            @plsc.parallel_loop(lo // _L, (lo + n) // _L, unroll=_UNROLL)
            def _scan(i):
                off = i * _L
                d = dst_v[pl.ds(off, _L)]
                s = src_v[pl.ds(off, _L)]
                w = w_v[pl.ds(off, _L)]
                plsc.addupdate_scatter(acc_v, [s], w, mask=(d == 0))

        cps0 = fetch(0, _CH0)

        zeros = jnp.zeros((_L,), jnp.float32)

        @plsc.parallel_loop(0, _N // _L, unroll=8)
        def _zero(i):
            acc_v[pl.ds(i * _L, _L)] = zeros

        for cp in cps0:
            cp.wait()
        cps1 = fetch(_CH0, _EPW - _CH0)
        scan(0, _CH0)
        for cp in cps1:
            cp.wait()
        scan(_CH0, _EPW - _CH0)

        pltpu.sync_copy(acc_v, out_hbm.at[wid])

    return k(ei_flat, wts)


def _tc_embed(x, w_fc, b_fc):
    def body(x_ref, wfc_ref, bfc_ref, y_ref):
        y = lax.dot_general(x_ref[...], wfc_ref[...],
                            (((1,), (1,)), ((), ())),
                            preferred_element_type=jnp.float32)
        y_ref[...] = jnp.maximum(y + bfc_ref[...], 0.0).astype(jnp.bfloat16)

    return pl.pallas_call(
        body,
        out_shape=jax.ShapeDtypeStruct((_N, _D), jnp.bfloat16),
    )(x, w_fc, b_fc.reshape(1, _D))


def _tc_final(y, wp, w_agg, b_agg):
    def body(y_ref, wp_ref, wagg_ref, bagg_ref, out_ref):
        wrow = jnp.sum(wp_ref[...], axis=0, keepdims=True)
        acc = lax.dot_general(wrow, y_ref[...].astype(jnp.float32),
                              (((1,), (0,)), ((), ())),
                              preferred_element_type=jnp.float32)
        r = jnp.maximum(acc, 0.0)
        out_ref[...] = lax.dot_general(r, wagg_ref[...],
                                       (((1,), (1,)), ((), ())),
                                       preferred_element_type=jnp.float32
                                       ) + bagg_ref[...]

    return pl.pallas_call(
        body,
        out_shape=jax.ShapeDtypeStruct((1, _D), jnp.float32),
    )(y, wp, w_agg, b_agg.reshape(1, _D))


def kernel(node_features, edge_index, edge_weights, W_fc, b_fc, W_agg, b_agg):
    ei = edge_index.astype(jnp.int32)
    wp = _sc_ego_weights(ei.reshape(2 * _E), edge_weights)
    y = _tc_embed(node_features, W_fc, b_fc)
    out = _tc_final(y, wp, W_agg, b_agg)
    return out.reshape(_D)


# speedup vs baseline: 1.0135x; 1.0135x over previous
import functools

import jax
import jax.numpy as jnp
from jax import lax
from jax.experimental import pallas as pl
from jax.experimental.pallas import tpu as pltpu
from jax.experimental.pallas import tpu_sc as plsc

_N = 10000
_E = 320000
_D = 128
_NC, _NS = 2, 16
_NW = _NC * _NS
_EPW = _E // _NW
_L = 16
_UNROLL = 5


def _sc_ego_weights(ei_flat, wts):
    mesh = plsc.VectorSubcoreMesh(
        core_axis_name="c", subcore_axis_name="s",
        num_cores=_NC, num_subcores=_NS)

    @functools.partial(
        pl.kernel,
        out_type=jax.ShapeDtypeStruct((_NW, _N), jnp.float32),
        mesh=mesh,
        compiler_params=pltpu.CompilerParams(needs_layout_passes=False),
        scratch_types=[
            pltpu.VMEM((_EPW,), jnp.int32),
            pltpu.VMEM((_EPW,), jnp.int32),
            pltpu.VMEM((_EPW,), jnp.float32),
            pltpu.VMEM((_N,), jnp.float32),
            pltpu.SemaphoreType.DMA,
            pltpu.SemaphoreType.DMA,
            pltpu.SemaphoreType.DMA,
        ],
    )
    def k(ei_hbm, w_hbm, out_hbm, src_v, dst_v, w_v, acc_v, s0, s1, s2):
        wid = lax.axis_index("s") * _NC + lax.axis_index("c")
        base = wid * _EPW
        cp_s = pltpu.async_copy(ei_hbm.at[pl.ds(base, _EPW)], src_v, s0)
        cp_d = pltpu.async_copy(ei_hbm.at[pl.ds(_E + base, _EPW)], dst_v, s1)
        cp_w = pltpu.async_copy(w_hbm.at[pl.ds(base, _EPW)], w_v, s2)

        zeros = jnp.zeros((_L,), jnp.float32)

        @plsc.parallel_loop(0, _N // _L, unroll=8)
        def _zero(i):
            acc_v[pl.ds(i * _L, _L)] = zeros

        cp_s.wait()
        cp_d.wait()
        cp_w.wait()

        @plsc.parallel_loop(0, _EPW // _L, unroll=_UNROLL)
        def _scan(i):
            off = i * _L
            d = dst_v[pl.ds(off, _L)]
            s = src_v[pl.ds(off, _L)]
            w = w_v[pl.ds(off, _L)]
            plsc.addupdate_scatter(acc_v, [s], w, mask=(d == 0))

        pltpu.sync_copy(acc_v, out_hbm.at[wid])

    return k(ei_flat, wts)


def _tc_embed(x, w_fc, b_fc):
    def body(x_ref, wfc_ref, bfc_ref, y_ref):
        y = lax.dot_general(x_ref[...], wfc_ref[...],
                            (((1,), (1,)), ((), ())),
                            preferred_element_type=jnp.float32)
        y_ref[...] = jnp.maximum(y + bfc_ref[...], 0.0).astype(jnp.bfloat16)

    return pl.pallas_call(
        body,
        out_shape=jax.ShapeDtypeStruct((_N, _D), jnp.bfloat16),
    )(x, w_fc, b_fc.reshape(1, _D))


def _tc_final(y, wp, w_agg, b_agg):
    def body(y_ref, wp_ref, wagg_ref, bagg_ref, out_ref):
        wrow = jnp.sum(wp_ref[...], axis=0, keepdims=True)
        acc = lax.dot_general(wrow, y_ref[...].astype(jnp.float32),
                              (((1,), (0,)), ((), ())),
                              preferred_element_type=jnp.float32)
        r = jnp.maximum(acc, 0.0)
        out_ref[...] = lax.dot_general(r, wagg_ref[...],
                                       (((1,), (1,)), ((), ())),
                                       preferred_element_type=jnp.float32
                                       ) + bagg_ref[...]

    return pl.pallas_call(
        body,
        out_shape=jax.ShapeDtypeStruct((1, _D), jnp.float32),
    )(y, wp, w_agg, b_agg.reshape(1, _D))


def kernel(node_features, edge_index, edge_weights, W_fc, b_fc, W_agg, b_agg):
    ei = edge_index.astype(jnp.int32)
    wp = _sc_ego_weights(ei.reshape(2 * _E), edge_weights)
    y = _tc_embed(node_features, W_fc, b_fc)
    out = _tc_final(y, wp, W_agg, b_agg)
    return out.reshape(_D)
